# baseline (speedup 1.0000x reference)
import jax
import jax.numpy as jnp
from jax import lax
from jax.experimental import pallas as pl
from jax.experimental.pallas import tpu as pltpu
from jax.experimental.pallas import tpu_sc as plsc

N = 8192
A = 8
D = 256
NC = 2
NS = 16
NW = NC * NS
ROWS_PER_W = N // NW
BLOCK = 256


def _sc_tile(e):
    mesh = plsc.ScalarSubcoreMesh(axis_name="core", num_cores=NC)

    @pl.kernel(
        out_type=jax.ShapeDtypeStruct((N, D), jnp.float32),
        mesh=mesh,
        scratch_types=[pltpu.VMEM_SHARED((BLOCK, D), jnp.float32),
                       pltpu.SemaphoreType.DMA],
    )
    def k(e_hbm, o_hbm, shared, sem):
        cid = lax.axis_index("core")
        half = BLOCK
        fills = [
            pltpu.async_copy(e_hbm, shared.at[pl.ds(s * A, A)], sem)
            for s in range(BLOCK // A)
        ]
        for c in fills:
            c.wait()
        writes = [
            pltpu.async_copy(
                shared,
                o_hbm.at[pl.ds((cid * (N // NC // half) + j) * half, half)],
                sem)
            for j in range(N // NC // half)
        ]
        for c in writes:
            c.wait()

    return k(e)


def _sc_tile_vector(e):
    mesh = plsc.VectorSubcoreMesh(core_axis_name="core",
                                  subcore_axis_name="subcore")

    @pl.kernel(
        out_type=jax.ShapeDtypeStruct((N, D), jnp.float32),
        mesh=mesh,
        scratch_types=[pltpu.VMEM_SHARED((ROWS_PER_W // 2, D), jnp.float32),
                       pltpu.SemaphoreType.DMA],
    )
    def k(e_hbm, o_hbm, shared, sem):
        sid = lax.axis_index("subcore")
        wid = sid * NC + lax.axis_index("core")
        base = wid * ROWS_PER_W
        half = ROWS_PER_W // 2
        pltpu.sync_copy(e_hbm, shared.at[pl.ds(sid * A, A)])
        plsc.subcore_barrier()
        c0 = pltpu.async_copy(shared, o_hbm.at[pl.ds(base, half)], sem)
        c1 = pltpu.async_copy(shared, o_hbm.at[pl.ds(base + half, half)], sem)
        c0.wait()
        c1.wait()

    return k(e)


def kernel(x, E_relative_position):
    del x
    return _sc_tile(E_relative_position)


# speedup vs baseline: 1.1198x; 1.1198x over previous
import jax
import jax.numpy as jnp
from jax import lax
from jax.experimental import pallas as pl
from jax.experimental.pallas import tpu as pltpu
from jax.experimental.pallas import tpu_sc as plsc

N = 8192
A = 8
D = 256
NC = 2
NS = 16
NW = NC * NS
ROWS_PER_W = N // NW
BLOCK = 64


def _sc_tile(e):
    mesh = plsc.ScalarSubcoreMesh(axis_name="core", num_cores=NC)

    @pl.kernel(
        out_type=jax.ShapeDtypeStruct((N, D), jnp.float32),
        mesh=mesh,
        scratch_types=[pltpu.VMEM_SHARED((BLOCK, D), jnp.float32),
                       pltpu.SemaphoreType.DMA],
    )
    def k(e_hbm, o_hbm, shared, sem):
        cid = lax.axis_index("core")
        half = BLOCK
        fills = [
            pltpu.async_copy(e_hbm, shared.at[pl.ds(s * A, A)], sem)
            for s in range(BLOCK // A)
        ]
        for c in fills:
            c.wait()
        writes = [
            pltpu.async_copy(
                shared,
                o_hbm.at[pl.ds((cid * (N // NC // half) + j) * half, half)],
                sem)
            for j in range(N // NC // half)
        ]
        for c in writes:
            c.wait()

    return k(e)


def _sc_tile_vector(e):
    mesh = plsc.VectorSubcoreMesh(core_axis_name="core",
                                  subcore_axis_name="subcore")

    @pl.kernel(
        out_type=jax.ShapeDtypeStruct((N, D), jnp.float32),
        mesh=mesh,
        scratch_types=[pltpu.VMEM_SHARED((ROWS_PER_W // 2, D), jnp.float32),
                       pltpu.SemaphoreType.DMA],
    )
    def k(e_hbm, o_hbm, shared, sem):
        sid = lax.axis_index("subcore")
        wid = sid * NC + lax.axis_index("core")
        base = wid * ROWS_PER_W
        half = ROWS_PER_W // 2
        pltpu.sync_copy(e_hbm, shared.at[pl.ds(sid * A, A)])
        plsc.subcore_barrier()
        c0 = pltpu.async_copy(shared, o_hbm.at[pl.ds(base, half)], sem)
        c1 = pltpu.async_copy(shared, o_hbm.at[pl.ds(base + half, half)], sem)
        c0.wait()
        c1.wait()

    return k(e)


def kernel(x, E_relative_position):
    del x
    return _sc_tile(E_relative_position)


# speedup vs baseline: 1.1235x; 1.0033x over previous
import jax
import jax.numpy as jnp
from jax import lax
from jax.experimental import pallas as pl
from jax.experimental.pallas import tpu as pltpu
from jax.experimental.pallas import tpu_sc as plsc

N = 8192
A = 8
D = 256
NC = 2
NS = 16
NW = NC * NS
ROWS_PER_W = N // NW
BLOCK = 32


def _sc_tile(e):
    mesh = plsc.ScalarSubcoreMesh(axis_name="core", num_cores=NC)

    @pl.kernel(
        out_type=jax.ShapeDtypeStruct((N, D), jnp.float32),
        mesh=mesh,
        scratch_types=[pltpu.VMEM_SHARED((BLOCK, D), jnp.float32),
                       pltpu.SemaphoreType.DMA],
    )
    def k(e_hbm, o_hbm, shared, sem):
        cid = lax.axis_index("core")
        half = BLOCK
        fills = [
            pltpu.async_copy(e_hbm, shared.at[pl.ds(s * A, A)], sem)
            for s in range(BLOCK // A)
        ]
        for c in fills:
            c.wait()
        writes = [
            pltpu.async_copy(
                shared,
                o_hbm.at[pl.ds((cid * (N // NC // half) + j) * half, half)],
                sem)
            for j in range(N // NC // half)
        ]
        for c in writes:
            c.wait()

    return k(e)


def _sc_tile_vector(e):
    mesh = plsc.VectorSubcoreMesh(core_axis_name="core",
                                  subcore_axis_name="subcore")

    @pl.kernel(
        out_type=jax.ShapeDtypeStruct((N, D), jnp.float32),
        mesh=mesh,
        scratch_types=[pltpu.VMEM_SHARED((ROWS_PER_W // 2, D), jnp.float32),
                       pltpu.SemaphoreType.DMA],
    )
    def k(e_hbm, o_hbm, shared, sem):
        sid = lax.axis_index("subcore")
        wid = sid * NC + lax.axis_index("core")
        base = wid * ROWS_PER_W
        half = ROWS_PER_W // 2
        pltpu.sync_copy(e_hbm, shared.at[pl.ds(sid * A, A)])
        plsc.subcore_barrier()
        c0 = pltpu.async_copy(shared, o_hbm.at[pl.ds(base, half)], sem)
        c1 = pltpu.async_copy(shared, o_hbm.at[pl.ds(base + half, half)], sem)
        c0.wait()
        c1.wait()

    return k(e)


def kernel(x, E_relative_position):
    del x
    return _sc_tile(E_relative_position)
